# Initial kernel scaffold; baseline (speedup 1.0000x reference)
import jax
import jax.numpy as jnp
from jax.experimental import pallas as pl


def kernel(obs_v, obs_w, rel_lang_embd, W_obs, b_obs, W_r1, b_r1, W_r2, b_r2, W_msg, W_upd, W_dec, b_dec):
    raise NotImplementedError("write your pallas kernel here")


# speedup vs baseline: 1.6445x; 1.6445x over previous
import functools

import jax
import jax.numpy as jnp
import numpy as np
from jax.experimental import pallas as pl

BS = 4
NL = 5
P = 32
D = 512
WD = 512
OUT = 7
HW = 160
K = HW * HW
ALPHA = 0.5
NUM_LAYERS = 3
G = BS * NL

KBLK = 3200


def _obs_gemm_kernel(v_ref, w_ref, b_ref, o_ref):
    k = pl.program_id(0)

    @pl.when(k == 0)
    def _():
        o_ref[...] = jnp.broadcast_to(b_ref[...], o_ref.shape)

    o_ref[...] += jnp.dot(v_ref[...], w_ref[...],
                          preferred_element_type=jnp.float32)


def _obs_gemm(v, W_obs, b_obs):
    n = v.shape[0]
    return pl.pallas_call(
        _obs_gemm_kernel,
        grid=(K // KBLK,),
        in_specs=[
            pl.BlockSpec((n, KBLK), lambda k: (0, k)),
            pl.BlockSpec((KBLK, D), lambda k: (k, 0)),
            pl.BlockSpec((1, D), lambda k: (0, 0)),
        ],
        out_specs=pl.BlockSpec((n, D), lambda k: (0, 0)),
        out_shape=jax.ShapeDtypeStruct((n, D), jnp.float32),
    )(v, W_obs, b_obs.reshape(1, D))


def _fused_kernel(w_ref, g_ref, h_all_ref, h0_ref, rel_ref,
                  wr1_ref, br1_ref, wr2_ref, br2_ref,
                  wmsg_ref, wupd_ref, wdec_ref, bdec_ref, o_ref):
    f32 = jnp.float32

    w = w_ref[...]
    logp = jax.nn.log_softmax(w, axis=-1)
    probs = ALPHA * jnp.exp(logp) + (1.0 - ALPHA) / P
    logq = jnp.log(probs)
    scores = logq[:, None, :] + g_ref[...]
    smax = jnp.max(scores, axis=-1, keepdims=True)
    iota = jax.lax.broadcasted_iota(jnp.int32, (G, P, P), 2)
    idx = jnp.min(jnp.where(scores >= smax, iota, P), axis=-1)
    onehot = (iota == idx[:, :, None]).astype(f32)

    delta = (logp - logq)[:, None, :]
    new_w = jnp.sum(onehot * delta, axis=-1)
    new_w = jax.nn.log_softmax(new_w, axis=-1)

    h_all = h_all_ref[...].reshape(G, P, D)
    h_obs_list = [jnp.dot(onehot[g], h_all[g], preferred_element_type=f32)
                  for g in range(G)]
    h_obs = jnp.stack(h_obs_list, axis=0)
    h_obs4 = h_obs.reshape(BS, NL, P, D)

    rel = rel_ref[...].reshape(BS * 4, WD)
    r = jax.nn.leaky_relu(
        jnp.dot(rel, wr1_ref[...], preferred_element_type=f32) + br1_ref[...])
    r = jnp.dot(r, wr2_ref[...], preferred_element_type=f32) + br2_ref[...]
    r = r.reshape(BS, 4, D)
    rel8 = jnp.stack([r, r], axis=2).reshape(BS, 8, D)

    wmsg_top = wmsg_ref[0:D, :]
    wmsg_bot = wmsg_ref[D:2 * D, :]
    wupd_top = wupd_ref[0:D, :]
    wupd_bot = wupd_ref[D:2 * D, :]

    rel_msg = jnp.dot(rel8.reshape(BS * 8, D), wmsg_bot,
                      preferred_element_type=f32).reshape(BS, 8, 1, D)

    obs_gate = jax.nn.softmax(new_w, axis=-1).reshape(BS * NL * P, 1)
    gated_obs = obs_gate * h_obs.reshape(BS * NL * P, D)

    h_v = h0_ref[...]
    for _ in range(NUM_LAYERS):
        hm = jnp.dot(h_v, wmsg_top, preferred_element_type=f32)
        hm4 = hm.reshape(BS, NL, P, D)
        m_in = [jax.nn.relu(hm4[:, e] + rel_msg[:, e]) for e in range(4)]
        m_out = [jax.nn.relu(hm4[:, NL - 1] + rel_msg[:, 4 + e])
                 for e in range(4)]
        agg_last = (m_in[0] + m_in[1] + m_in[2] + m_in[3]) * 0.25
        agg = jnp.stack(m_out + [agg_last], axis=1)
        upd = (jnp.dot(h_v, wupd_top, preferred_element_type=f32)
               + jnp.dot(agg.reshape(BS * NL * P, D), wupd_bot,
                         preferred_element_type=f32))
        h_v = jnp.tanh(upd) + gated_obs

    hv4 = h_v.reshape(BS, NL, P, D)
    nw4 = new_w.reshape(BS, NL, P)
    part_w = jax.nn.softmax(nw4[:, NL - 1], axis=-1)
    h_last = hv4[:, NL - 1]
    h_out = jnp.stack(
        [jnp.dot(part_w[b:b + 1], h_last[b],
                 preferred_element_type=f32)[0] for b in range(BS)], axis=0)
    o_ref[...] = jnp.dot(h_out, wdec_ref[...],
                         preferred_element_type=f32) + bdec_ref[...]


def _fused(w2, gumbel, h_all, h0, rel_lang_embd,
           W_r1, b_r1, W_r2, b_r2, W_msg, W_upd, W_dec, b_dec):
    full = lambda a: pl.BlockSpec(a.shape, lambda: tuple([0] * a.ndim))
    args = (w2, gumbel, h_all, h0, rel_lang_embd,
            W_r1, b_r1.reshape(1, 2 * D), W_r2, b_r2.reshape(1, D),
            W_msg, W_upd, W_dec, b_dec.reshape(1, OUT))
    return pl.pallas_call(
        _fused_kernel,
        in_specs=[full(a) for a in args],
        out_specs=pl.BlockSpec((BS, OUT), lambda: (0, 0)),
        out_shape=jax.ShapeDtypeStruct((BS, OUT), jnp.float32),
    )(*args)


@jax.jit
def kernel(obs_v, obs_w, rel_lang_embd, W_obs, b_obs, W_r1, b_r1, W_r2, b_r2,
           W_msg, W_upd, W_dec, b_dec):
    v = obs_v.reshape(BS * NL * P, K)
    w2 = obs_w.reshape(G, P)

    gumbel = jax.random.gumbel(jax.random.key(42), (G, P, P), jnp.float32)
    h0 = 0.2 * jax.random.normal(jax.random.key(7), (BS, NL, P, D),
                                 dtype=jnp.float32)

    h_all = _obs_gemm(v, W_obs, b_obs)
    return _fused(w2, gumbel, h_all, h0.reshape(BS * NL * P, D),
                  rel_lang_embd, W_r1, b_r1, W_r2, b_r2,
                  W_msg, W_upd, W_dec, b_dec)


# speedup vs baseline: 1.7856x; 1.0858x over previous
import functools

import jax
import jax.numpy as jnp
import numpy as np
from jax.experimental import pallas as pl

BS = 4
NL = 5
P = 32
D = 512
WD = 512
OUT = 7
HW = 160
K = HW * HW
ALPHA = 0.5
NUM_LAYERS = 3
G = BS * NL

KBLK = 3200

_GUMBEL = np.asarray(jax.random.gumbel(
    jax.random.key(42), (BS * NL, P, P), jnp.float32))
_H0 = np.asarray(0.2 * jax.random.normal(
    jax.random.key(7), (BS, NL, P, D), dtype=jnp.float32)).reshape(
        BS * NL * P, D)


def _obs_gemm_kernel(v_ref, w_ref, b_ref, o_ref):
    k = pl.program_id(0)

    @pl.when(k == 0)
    def _():
        o_ref[...] = jnp.broadcast_to(b_ref[...], o_ref.shape)

    o_ref[...] += jnp.dot(v_ref[...], w_ref[...],
                          preferred_element_type=jnp.float32)


def _obs_gemm(v, W_obs, b_obs):
    n = v.shape[0]
    return pl.pallas_call(
        _obs_gemm_kernel,
        grid=(K // KBLK,),
        in_specs=[
            pl.BlockSpec((n, KBLK), lambda k: (0, k)),
            pl.BlockSpec((KBLK, D), lambda k: (k, 0)),
            pl.BlockSpec((1, D), lambda k: (0, 0)),
        ],
        out_specs=pl.BlockSpec((n, D), lambda k: (0, 0)),
        out_shape=jax.ShapeDtypeStruct((n, D), jnp.float32),
    )(v, W_obs, b_obs.reshape(1, D))


def _fused_kernel(w_ref, g_ref, h_all_ref, h0_ref, rel_ref,
                  wr1_ref, br1_ref, wr2_ref, br2_ref,
                  wmsg_ref, wupd_ref, wdec_ref, bdec_ref, o_ref):
    f32 = jnp.float32

    w = w_ref[...]
    logp = jax.nn.log_softmax(w, axis=-1)
    probs = ALPHA * jnp.exp(logp) + (1.0 - ALPHA) / P
    logq = jnp.log(probs)
    scores = logq[:, None, :] + g_ref[...]
    smax = jnp.max(scores, axis=-1, keepdims=True)
    iota = jax.lax.broadcasted_iota(jnp.int32, (G, P, P), 2)
    idx = jnp.min(jnp.where(scores >= smax, iota, P), axis=-1)
    onehot = (iota == idx[:, :, None]).astype(f32)

    delta = (logp - logq)[:, None, :]
    new_w = jnp.sum(onehot * delta, axis=-1)
    new_w = jax.nn.log_softmax(new_w, axis=-1)

    h_all = h_all_ref[...].reshape(G, P, D)
    h_obs_list = [jnp.dot(onehot[g], h_all[g], preferred_element_type=f32)
                  for g in range(G)]
    h_obs = jnp.stack(h_obs_list, axis=0)
    h_obs4 = h_obs.reshape(BS, NL, P, D)

    rel = rel_ref[...].reshape(BS * 4, WD)
    r = jax.nn.leaky_relu(
        jnp.dot(rel, wr1_ref[...], preferred_element_type=f32) + br1_ref[...])
    r = jnp.dot(r, wr2_ref[...], preferred_element_type=f32) + br2_ref[...]
    r = r.reshape(BS, 4, D)
    rel8 = jnp.stack([r, r], axis=2).reshape(BS, 8, D)

    wmsg_top = wmsg_ref[0:D, :]
    wmsg_bot = wmsg_ref[D:2 * D, :]
    wupd_top = wupd_ref[0:D, :]
    wupd_bot = wupd_ref[D:2 * D, :]

    rel_msg = jnp.dot(rel8.reshape(BS * 8, D), wmsg_bot,
                      preferred_element_type=f32).reshape(BS, 8, 1, D)

    obs_gate = jax.nn.softmax(new_w, axis=-1).reshape(BS * NL * P, 1)
    gated_obs = obs_gate * h_obs.reshape(BS * NL * P, D)

    h_v = h0_ref[...]
    for _ in range(NUM_LAYERS):
        hm = jnp.dot(h_v, wmsg_top, preferred_element_type=f32)
        hm4 = hm.reshape(BS, NL, P, D)
        m_in = [jax.nn.relu(hm4[:, e] + rel_msg[:, e]) for e in range(4)]
        m_out = [jax.nn.relu(hm4[:, NL - 1] + rel_msg[:, 4 + e])
                 for e in range(4)]
        agg_last = (m_in[0] + m_in[1] + m_in[2] + m_in[3]) * 0.25
        agg = jnp.stack(m_out + [agg_last], axis=1)
        upd = (jnp.dot(h_v, wupd_top, preferred_element_type=f32)
               + jnp.dot(agg.reshape(BS * NL * P, D), wupd_bot,
                         preferred_element_type=f32))
        h_v = jnp.tanh(upd) + gated_obs

    hv4 = h_v.reshape(BS, NL, P, D)
    nw4 = new_w.reshape(BS, NL, P)
    part_w = jax.nn.softmax(nw4[:, NL - 1], axis=-1)
    h_last = hv4[:, NL - 1]
    h_out = jnp.stack(
        [jnp.dot(part_w[b:b + 1], h_last[b],
                 preferred_element_type=f32)[0] for b in range(BS)], axis=0)
    o_ref[...] = jnp.dot(h_out, wdec_ref[...],
                         preferred_element_type=f32) + bdec_ref[...]


def _fused(w2, gumbel, h_all, h0, rel_lang_embd,
           W_r1, b_r1, W_r2, b_r2, W_msg, W_upd, W_dec, b_dec):
    full = lambda a: pl.BlockSpec(a.shape, lambda: tuple([0] * a.ndim))
    args = (w2, gumbel, h_all, h0, rel_lang_embd,
            W_r1, b_r1.reshape(1, 2 * D), W_r2, b_r2.reshape(1, D),
            W_msg, W_upd, W_dec, b_dec.reshape(1, OUT))
    return pl.pallas_call(
        _fused_kernel,
        in_specs=[full(a) for a in args],
        out_specs=pl.BlockSpec((BS, OUT), lambda: (0, 0)),
        out_shape=jax.ShapeDtypeStruct((BS, OUT), jnp.float32),
    )(*args)


@jax.jit
def kernel(obs_v, obs_w, rel_lang_embd, W_obs, b_obs, W_r1, b_r1, W_r2, b_r2,
           W_msg, W_upd, W_dec, b_dec):
    v = obs_v.reshape(BS * NL * P, K)
    w2 = obs_w.reshape(G, P)

    h_all = _obs_gemm(v, W_obs, b_obs)
    return _fused(w2, jnp.asarray(_GUMBEL), h_all, jnp.asarray(_H0),
                  rel_lang_embd, W_r1, b_r1, W_r2, b_r2,
                  W_msg, W_upd, W_dec, b_dec)


# speedup vs baseline: 3.8991x; 2.1836x over previous
import functools

import jax
import jax.numpy as jnp
import numpy as np
from jax.experimental import pallas as pl

BS = 4
NL = 5
P = 32
D = 512
WD = 512
OUT = 7
HW = 160
K = HW * HW
ALPHA = 0.5
NUM_LAYERS = 3
G = BS * NL

KBLK = 5120

def _rng_consts():
    g = jax.random.gumbel(jax.random.key(42), (BS * NL, P, P), jnp.float32)
    h0 = 0.2 * jax.random.normal(jax.random.key(7), (BS, NL, P, D),
                                 dtype=jnp.float32)
    return g, h0.reshape(BS * NL * P, D)


try:
    _GUMBEL, _H0 = (np.asarray(x) for x in _rng_consts())
except Exception:
    _GUMBEL, _H0 = None, None


KROWS = 8


def _obs_gemm_kernel(v_ref, w_ref, b_ref, o_ref):
    k = pl.program_id(0)

    @pl.when(k == 0)
    def _():
        o_ref[...] = jnp.broadcast_to(b_ref[...], o_ref.shape)

    v = v_ref[...].reshape(BS * NL * P, KROWS * HW)
    o_ref[...] += jnp.dot(v, w_ref[...], preferred_element_type=jnp.float32)


def _obs_gemm(obs_v, W_obs, b_obs):
    n = BS * NL * P
    return pl.pallas_call(
        _obs_gemm_kernel,
        grid=(HW // KROWS,),
        in_specs=[
            pl.BlockSpec((BS, NL * P, KROWS, HW), lambda k: (0, 0, k, 0)),
            pl.BlockSpec((KROWS * HW, D), lambda k: (k, 0)),
            pl.BlockSpec((1, D), lambda k: (0, 0)),
        ],
        out_specs=pl.BlockSpec((n, D), lambda k: (0, 0)),
        out_shape=jax.ShapeDtypeStruct((n, D), jnp.float32),
    )(obs_v, W_obs, b_obs.reshape(1, D))


def _fused_kernel(w_ref, g_ref, h_all_ref, h0_ref, rel_ref,
                  wr1_ref, br1_ref, wr2_ref, br2_ref,
                  wmsg_ref, wupd_ref, wdec_ref, bdec_ref, o_ref):
    f32 = jnp.float32

    w = w_ref[...]
    logp = jax.nn.log_softmax(w, axis=-1)
    probs = ALPHA * jnp.exp(logp) + (1.0 - ALPHA) / P
    logq = jnp.log(probs)
    scores = logq[:, None, :] + g_ref[...]
    smax = jnp.max(scores, axis=-1, keepdims=True)
    iota = jax.lax.broadcasted_iota(jnp.int32, (G, P, P), 2)
    idx = jnp.min(jnp.where(scores >= smax, iota, P), axis=-1)
    onehot = (iota == idx[:, :, None]).astype(f32)

    delta = (logp - logq)[:, None, :]
    new_w = jnp.sum(onehot * delta, axis=-1)
    new_w = jax.nn.log_softmax(new_w, axis=-1)

    h_all = h_all_ref[...].reshape(G, P, D)
    h_obs_list = [jnp.dot(onehot[g], h_all[g], preferred_element_type=f32)
                  for g in range(G)]
    h_obs = jnp.stack(h_obs_list, axis=0)
    h_obs4 = h_obs.reshape(BS, NL, P, D)

    rel = rel_ref[...].reshape(BS * 4, WD)
    r = jax.nn.leaky_relu(
        jnp.dot(rel, wr1_ref[...], preferred_element_type=f32) + br1_ref[...])
    r = jnp.dot(r, wr2_ref[...], preferred_element_type=f32) + br2_ref[...]
    r = r.reshape(BS, 4, D)
    rel8 = jnp.stack([r, r], axis=2).reshape(BS, 8, D)

    wmsg_top = wmsg_ref[0:D, :]
    wmsg_bot = wmsg_ref[D:2 * D, :]
    wupd_top = wupd_ref[0:D, :]
    wupd_bot = wupd_ref[D:2 * D, :]

    rel_msg = jnp.dot(rel8.reshape(BS * 8, D), wmsg_bot,
                      preferred_element_type=f32).reshape(BS, 8, 1, D)

    obs_gate = jax.nn.softmax(new_w, axis=-1).reshape(BS * NL * P, 1)
    gated_obs = obs_gate * h_obs.reshape(BS * NL * P, D)

    h_v = h0_ref[...]
    for _ in range(NUM_LAYERS):
        hm = jnp.dot(h_v, wmsg_top, preferred_element_type=f32)
        hm4 = hm.reshape(BS, NL, P, D)
        m_in = [jax.nn.relu(hm4[:, e] + rel_msg[:, e]) for e in range(4)]
        m_out = [jax.nn.relu(hm4[:, NL - 1] + rel_msg[:, 4 + e])
                 for e in range(4)]
        agg_last = (m_in[0] + m_in[1] + m_in[2] + m_in[3]) * 0.25
        agg = jnp.stack(m_out + [agg_last], axis=1)
        upd = (jnp.dot(h_v, wupd_top, preferred_element_type=f32)
               + jnp.dot(agg.reshape(BS * NL * P, D), wupd_bot,
                         preferred_element_type=f32))
        h_v = jnp.tanh(upd) + gated_obs

    hv4 = h_v.reshape(BS, NL, P, D)
    nw4 = new_w.reshape(BS, NL, P)
    part_w = jax.nn.softmax(nw4[:, NL - 1], axis=-1)
    h_last = hv4[:, NL - 1]
    h_out = jnp.stack(
        [jnp.dot(part_w[b:b + 1], h_last[b],
                 preferred_element_type=f32)[0] for b in range(BS)], axis=0)
    o_ref[...] = jnp.dot(h_out, wdec_ref[...],
                         preferred_element_type=f32) + bdec_ref[...]


def _fused(w2, gumbel, h_all, h0, rel_lang_embd,
           W_r1, b_r1, W_r2, b_r2, W_msg, W_upd, W_dec, b_dec):
    full = lambda a: pl.BlockSpec(a.shape, lambda: tuple([0] * a.ndim))
    args = (w2, gumbel, h_all, h0, rel_lang_embd,
            W_r1, b_r1.reshape(1, 2 * D), W_r2, b_r2.reshape(1, D),
            W_msg, W_upd, W_dec, b_dec.reshape(1, OUT))
    return pl.pallas_call(
        _fused_kernel,
        in_specs=[full(a) for a in args],
        out_specs=pl.BlockSpec((BS, OUT), lambda: (0, 0)),
        out_shape=jax.ShapeDtypeStruct((BS, OUT), jnp.float32),
    )(*args)


@jax.jit
def kernel(obs_v, obs_w, rel_lang_embd, W_obs, b_obs, W_r1, b_r1, W_r2, b_r2,
           W_msg, W_upd, W_dec, b_dec):
    w2 = obs_w.reshape(G, P)

    if _GUMBEL is not None:
        gum, h0 = jnp.asarray(_GUMBEL), jnp.asarray(_H0)
    else:
        gum, h0 = _rng_consts()

    h_all = _obs_gemm(obs_v, W_obs, b_obs)
    return _fused(w2, gum, h_all, h0,
                  rel_lang_embd, W_r1, b_r1, W_r2, b_r2,
                  W_msg, W_upd, W_dec, b_dec)


# speedup vs baseline: 4.0334x; 1.0344x over previous
import jax
import jax.numpy as jnp
import numpy as np
from jax.experimental import pallas as pl
from jax.experimental.pallas import tpu as pltpu

BS = 4
NL = 5
P = 32
D = 512
WD = 512
OUT = 7
HW = 160
ALPHA = 0.5
NUM_LAYERS = 3
G = BS * NL
N = BS * NL * P
KROWS = 8


def _rng_consts():
    g = jax.random.gumbel(jax.random.key(42), (G, P, P), jnp.float32)
    h0 = 0.2 * jax.random.normal(jax.random.key(7), (BS, NL, P, D),
                                 dtype=jnp.float32)
    return g, h0.reshape(N, D)


try:
    _GUMBEL, _H0 = (np.asarray(x) for x in _rng_consts())
except Exception:
    _GUMBEL, _H0 = None, None


def _tail(w, gum, h_all, h0, rel, wr1, br1, wr2, br2, wmsg, wupd, wdec, bdec):
    f32 = jnp.float32

    logp = jax.nn.log_softmax(w, axis=-1)
    probs = ALPHA * jnp.exp(logp) + (1.0 - ALPHA) / P
    logq = jnp.log(probs)
    scores = logq[:, None, :] + gum
    smax = jnp.max(scores, axis=-1, keepdims=True)
    iota = jax.lax.broadcasted_iota(jnp.int32, (G, P, P), 2)
    idx = jnp.min(jnp.where(scores >= smax, iota, P), axis=-1)
    onehot = (iota == idx[:, :, None]).astype(f32)

    delta = (logp - logq)[:, None, :]
    new_w = jnp.sum(onehot * delta, axis=-1)
    new_w = jax.nn.log_softmax(new_w, axis=-1)

    h_all3 = h_all.reshape(G, P, D)
    h_obs = jnp.stack(
        [jnp.dot(onehot[g], h_all3[g], preferred_element_type=f32)
         for g in range(G)], axis=0)

    r = jax.nn.leaky_relu(
        jnp.dot(rel, wr1, preferred_element_type=f32) + br1)
    r = jnp.dot(r, wr2, preferred_element_type=f32) + br2
    r = r.reshape(BS, 4, D)
    rel8 = jnp.stack([r, r], axis=2).reshape(BS, 8, D)

    wmsg_top, wmsg_bot = wmsg[0:D, :], wmsg[D:2 * D, :]
    wupd_top, wupd_bot = wupd[0:D, :], wupd[D:2 * D, :]

    rel_msg = jnp.dot(rel8.reshape(BS * 8, D), wmsg_bot,
                      preferred_element_type=f32).reshape(BS, 8, 1, D)

    obs_gate = jax.nn.softmax(new_w, axis=-1).reshape(N, 1)
    gated_obs = obs_gate * h_obs.reshape(N, D)

    h_v = h0
    for _ in range(NUM_LAYERS):
        hm = jnp.dot(h_v, wmsg_top, preferred_element_type=f32)
        hm4 = hm.reshape(BS, NL, P, D)
        m_in = [jax.nn.relu(hm4[:, e] + rel_msg[:, e]) for e in range(4)]
        m_out = [jax.nn.relu(hm4[:, NL - 1] + rel_msg[:, 4 + e])
                 for e in range(4)]
        agg_last = (m_in[0] + m_in[1] + m_in[2] + m_in[3]) * 0.25
        agg = jnp.stack(m_out + [agg_last], axis=1)
        upd = (jnp.dot(h_v, wupd_top, preferred_element_type=f32)
               + jnp.dot(agg.reshape(N, D), wupd_bot,
                         preferred_element_type=f32))
        h_v = jnp.tanh(upd) + gated_obs

    hv4 = h_v.reshape(BS, NL, P, D)
    nw4 = new_w.reshape(BS, NL, P)
    part_w = jax.nn.softmax(nw4[:, NL - 1], axis=-1)
    h_last = hv4[:, NL - 1]
    h_out = jnp.stack(
        [jnp.dot(part_w[b:b + 1], h_last[b],
                 preferred_element_type=f32)[0] for b in range(BS)], axis=0)
    return jnp.dot(h_out, wdec, preferred_element_type=f32) + bdec


def _mega_kernel(v_ref, wobs_ref, bobs_ref, w2_ref, gum_ref, h0_ref, rel_ref,
                 wr1_ref, br1_ref, wr2_ref, br2_ref, wmsg_ref, wupd_ref,
                 wdec_ref, bdec_ref, o_ref, acc_ref):
    k = pl.program_id(0)

    @pl.when(k == 0)
    def _():
        acc_ref[...] = jnp.broadcast_to(bobs_ref[...], acc_ref.shape)

    v = v_ref[...].reshape(N, KROWS * HW)
    acc_ref[...] += jnp.dot(v, wobs_ref[...],
                            preferred_element_type=jnp.float32)

    @pl.when(k == pl.num_programs(0) - 1)
    def _():
        o_ref[...] = _tail(
            w2_ref[...], gum_ref[...], acc_ref[...], h0_ref[...],
            rel_ref[...].reshape(BS * 4, WD),
            wr1_ref[...], br1_ref[...], wr2_ref[...], br2_ref[...],
            wmsg_ref[...], wupd_ref[...], wdec_ref[...], bdec_ref[...])


@jax.jit
def kernel(obs_v, obs_w, rel_lang_embd, W_obs, b_obs, W_r1, b_r1, W_r2, b_r2,
           W_msg, W_upd, W_dec, b_dec):
    if _GUMBEL is not None:
        gum, h0 = jnp.asarray(_GUMBEL), jnp.asarray(_H0)
    else:
        gum, h0 = _rng_consts()

    const = lambda a: pl.BlockSpec(a.shape, lambda k: tuple([0] * a.ndim))
    args = (W_obs, b_obs.reshape(1, D), obs_w.reshape(G, P), gum, h0,
            rel_lang_embd, W_r1, b_r1.reshape(1, 2 * D), W_r2,
            b_r2.reshape(1, D), W_msg, W_upd, W_dec, b_dec.reshape(1, OUT))
    specs = [pl.BlockSpec((KROWS * HW, D), lambda k: (k, 0))]
    specs += [const(a) for a in args[1:]]
    return pl.pallas_call(
        _mega_kernel,
        grid=(HW // KROWS,),
        in_specs=[pl.BlockSpec((BS, NL * P, KROWS, HW),
                               lambda k: (0, 0, k, 0))] + specs,
        out_specs=pl.BlockSpec((BS, OUT), lambda k: (0, 0)),
        out_shape=jax.ShapeDtypeStruct((BS, OUT), jnp.float32),
        scratch_shapes=[pltpu.VMEM((N, D), jnp.float32)],
    )(obs_v, *args)


# speedup vs baseline: 4.3194x; 1.0709x over previous
import jax
import jax.numpy as jnp
import numpy as np
from jax.experimental import pallas as pl
from jax.experimental.pallas import tpu as pltpu

BS = 4
NL = 5
P = 32
D = 512
WD = 512
OUT = 7
HW = 160
ALPHA = 0.5
NUM_LAYERS = 3
G = BS * NL
N = BS * NL * P
KROWS = 16


def _rng_consts():
    g = jax.random.gumbel(jax.random.key(42), (G, P, P), jnp.float32)
    h0 = 0.2 * jax.random.normal(jax.random.key(7), (BS, NL, P, D),
                                 dtype=jnp.float32)
    return g, h0.reshape(N, D)


try:
    _GUMBEL, _H0 = (np.asarray(x) for x in _rng_consts())
except Exception:
    _GUMBEL, _H0 = None, None


def _tail(w, gum, h_all, h0, rel, wr1, br1, wr2, br2, wmsg, wupd, wdec, bdec):
    f32 = jnp.float32

    logp = jax.nn.log_softmax(w, axis=-1)
    probs = ALPHA * jnp.exp(logp) + (1.0 - ALPHA) / P
    logq = jnp.log(probs)
    scores = logq[:, None, :] + gum
    smax = jnp.max(scores, axis=-1, keepdims=True)
    iota = jax.lax.broadcasted_iota(jnp.int32, (G, P, P), 2)
    idx = jnp.min(jnp.where(scores >= smax, iota, P), axis=-1)
    onehot = (iota == idx[:, :, None]).astype(f32)

    delta = (logp - logq)[:, None, :]
    new_w = jnp.sum(onehot * delta, axis=-1)
    new_w = jax.nn.log_softmax(new_w, axis=-1)

    h_all3 = h_all.reshape(G, P, D)
    h_obs = jnp.stack(
        [jnp.dot(onehot[g], h_all3[g], preferred_element_type=f32)
         for g in range(G)], axis=0)

    r = jax.nn.leaky_relu(
        jnp.dot(rel, wr1, preferred_element_type=f32) + br1)
    r = jnp.dot(r, wr2, preferred_element_type=f32) + br2
    r = r.reshape(BS, 4, D)
    rel8 = jnp.stack([r, r], axis=2).reshape(BS, 8, D)

    wmsg_top, wmsg_bot = wmsg[0:D, :], wmsg[D:2 * D, :]
    wupd_top, wupd_bot = wupd[0:D, :], wupd[D:2 * D, :]

    rel_msg = jnp.dot(rel8.reshape(BS * 8, D), wmsg_bot,
                      preferred_element_type=f32).reshape(BS, 8, 1, D)

    obs_gate = jax.nn.softmax(new_w, axis=-1).reshape(N, 1)
    gated_obs = obs_gate * h_obs.reshape(N, D)

    h_v = h0
    for _ in range(NUM_LAYERS):
        hm = jnp.dot(h_v, wmsg_top, preferred_element_type=f32)
        hm4 = hm.reshape(BS, NL, P, D)
        m_in = [jax.nn.relu(hm4[:, e] + rel_msg[:, e]) for e in range(4)]
        m_out = [jax.nn.relu(hm4[:, NL - 1] + rel_msg[:, 4 + e])
                 for e in range(4)]
        agg_last = (m_in[0] + m_in[1] + m_in[2] + m_in[3]) * 0.25
        agg = jnp.stack(m_out + [agg_last], axis=1)
        upd = (jnp.dot(h_v, wupd_top, preferred_element_type=f32)
               + jnp.dot(agg.reshape(N, D), wupd_bot,
                         preferred_element_type=f32))
        h_v = jnp.tanh(upd) + gated_obs

    hv4 = h_v.reshape(BS, NL, P, D)
    nw4 = new_w.reshape(BS, NL, P)
    part_w = jax.nn.softmax(nw4[:, NL - 1], axis=-1)
    h_last = hv4[:, NL - 1]
    h_out = jnp.stack(
        [jnp.dot(part_w[b:b + 1], h_last[b],
                 preferred_element_type=f32)[0] for b in range(BS)], axis=0)
    return jnp.dot(h_out, wdec, preferred_element_type=f32) + bdec


def _mega_kernel(v_ref, wobs_ref, bobs_ref, w2_ref, gum_ref, h0_ref, rel_ref,
                 wr1_ref, br1_ref, wr2_ref, br2_ref, wmsg_ref, wupd_ref,
                 wdec_ref, bdec_ref, o_ref, acc_ref):
    k = pl.program_id(0)

    @pl.when(k == 0)
    def _():
        acc_ref[...] = jnp.broadcast_to(bobs_ref[...], acc_ref.shape)

    v = v_ref[...].reshape(N, KROWS * HW)
    acc_ref[...] += jnp.dot(v, wobs_ref[...],
                            preferred_element_type=jnp.float32)

    @pl.when(k == pl.num_programs(0) - 1)
    def _():
        o_ref[...] = _tail(
            w2_ref[...], gum_ref[...], acc_ref[...], h0_ref[...],
            rel_ref[...].reshape(BS * 4, WD),
            wr1_ref[...], br1_ref[...], wr2_ref[...], br2_ref[...],
            wmsg_ref[...], wupd_ref[...], wdec_ref[...], bdec_ref[...])


@jax.jit
def kernel(obs_v, obs_w, rel_lang_embd, W_obs, b_obs, W_r1, b_r1, W_r2, b_r2,
           W_msg, W_upd, W_dec, b_dec):
    if _GUMBEL is not None:
        gum, h0 = jnp.asarray(_GUMBEL), jnp.asarray(_H0)
    else:
        gum, h0 = _rng_consts()

    const = lambda a: pl.BlockSpec(a.shape, lambda k: tuple([0] * a.ndim))
    args = (W_obs, b_obs.reshape(1, D), obs_w.reshape(G, P), gum, h0,
            rel_lang_embd, W_r1, b_r1.reshape(1, 2 * D), W_r2,
            b_r2.reshape(1, D), W_msg, W_upd, W_dec, b_dec.reshape(1, OUT))
    specs = [pl.BlockSpec((KROWS * HW, D), lambda k: (k, 0))]
    specs += [const(a) for a in args[1:]]
    return pl.pallas_call(
        _mega_kernel,
        grid=(HW // KROWS,),
        in_specs=[pl.BlockSpec((BS, NL * P, KROWS, HW),
                               lambda k: (0, 0, k, 0))] + specs,
        out_specs=pl.BlockSpec((BS, OUT), lambda k: (0, 0)),
        out_shape=jax.ShapeDtypeStruct((BS, OUT), jnp.float32),
        scratch_shapes=[pltpu.VMEM((N, D), jnp.float32)],
    )(obs_v, *args)
